# Initial kernel scaffold; baseline (speedup 1.0000x reference)
import jax
import jax.numpy as jnp
from jax.experimental import pallas as pl


def kernel(hidden_states, gate_w, Wg, Wu, Wd, Sg, Su, Sd, shared_gate_w):
    raise NotImplementedError("write your pallas kernel here")


# speedup vs baseline: 1.5630x; 1.5630x over previous
import jax
import jax.numpy as jnp
from jax.experimental import pallas as pl
from jax.experimental.pallas import tpu as pltpu

T, D, E, DFF, DSH = 2048, 768, 64, 256, 512
TB = 512
NTB = T // TB


def _moe_body(x_ref, gate_w_ref, Wg_ref, Wu_ref, Wd_ref, Sg_ref, Su_ref,
              Sd_ref, sgw_ref, out_ref, combine_ref, acc_ref, xbf_ref):
    e = pl.program_id(0)
    tb = pl.program_id(1)
    sl = pl.ds(tb * TB, TB)

    @pl.when(e == 0)
    def _init():
        x = x_ref[sl, :]
        logits = jnp.dot(x, gate_w_ref[...],
                         preferred_element_type=jnp.float32)
        mx = jnp.max(logits, axis=-1, keepdims=True)
        p = jnp.exp(logits - mx)
        probs = p / jnp.sum(p, axis=-1, keepdims=True)
        m1 = jnp.max(probs, axis=-1, keepdims=True)
        m2 = jnp.max(jnp.where(probs == m1, -jnp.inf, probs), axis=-1,
                     keepdims=True)
        combine_ref[sl, :] = jnp.where(probs >= m2, probs / (m1 + m2), 0.0)

        xb = x.astype(jnp.bfloat16)
        xbf_ref[sl, :] = xb
        g = jnp.dot(xb, Sg_ref[...].astype(jnp.bfloat16),
                    preferred_element_type=jnp.float32)
        u = jnp.dot(xb, Su_ref[...].astype(jnp.bfloat16),
                    preferred_element_type=jnp.float32)
        h = (g * jax.nn.sigmoid(g) * u).astype(jnp.bfloat16)
        sh = jnp.dot(h, Sd_ref[...].astype(jnp.bfloat16),
                     preferred_element_type=jnp.float32)
        sgate = jax.nn.sigmoid(jnp.dot(x, sgw_ref[...],
                                       preferred_element_type=jnp.float32,
                                       precision=jax.lax.Precision.HIGHEST))
        acc_ref[sl, :] = sgate * sh

    xb = xbf_ref[sl, :]
    g = jnp.dot(xb, Wg_ref[0].astype(jnp.bfloat16),
                preferred_element_type=jnp.float32)
    u = jnp.dot(xb, Wu_ref[0].astype(jnp.bfloat16),
                preferred_element_type=jnp.float32)
    h = (g * jax.nn.sigmoid(g) * u).astype(jnp.bfloat16)
    eo = jnp.dot(h, Wd_ref[0].astype(jnp.bfloat16),
                 preferred_element_type=jnp.float32)
    onehot = (jax.lax.broadcasted_iota(jnp.int32, (E, 1), 0) == e
              ).astype(jnp.float32)
    w = jnp.dot(combine_ref[sl, :], onehot,
                preferred_element_type=jnp.float32)
    acc_ref[sl, :] += w * eo

    @pl.when(e == E - 1)
    def _fin():
        out_ref[sl, :] = acc_ref[sl, :]


def kernel(hidden_states, gate_w, Wg, Wu, Wd, Sg, Su, Sd, shared_gate_w):
    return pl.pallas_call(
        _moe_body,
        grid=(E, NTB),
        in_specs=[
            pl.BlockSpec((T, D), lambda e, t: (0, 0)),
            pl.BlockSpec((D, E), lambda e, t: (0, 0)),
            pl.BlockSpec((1, D, DFF), lambda e, t: (e, 0, 0)),
            pl.BlockSpec((1, D, DFF), lambda e, t: (e, 0, 0)),
            pl.BlockSpec((1, DFF, D), lambda e, t: (e, 0, 0)),
            pl.BlockSpec((D, DSH), lambda e, t: (0, 0)),
            pl.BlockSpec((D, DSH), lambda e, t: (0, 0)),
            pl.BlockSpec((DSH, D), lambda e, t: (0, 0)),
            pl.BlockSpec((D, 1), lambda e, t: (0, 0)),
        ],
        out_specs=pl.BlockSpec((T, D), lambda e, t: (0, 0)),
        out_shape=jax.ShapeDtypeStruct((T, D), jnp.float32),
        scratch_shapes=[
            pltpu.VMEM((T, E), jnp.float32),
            pltpu.VMEM((T, D), jnp.float32),
            pltpu.VMEM((T, D), jnp.bfloat16),
        ],
        compiler_params=pltpu.CompilerParams(
            dimension_semantics=("arbitrary", "arbitrary")),
    )(hidden_states, gate_w, Wg, Wu, Wd, Sg, Su, Sd, shared_gate_w)
